# baseline (speedup 1.0000x reference)
import functools

import jax
import jax.numpy as jnp
from jax.experimental import pallas as pl


def _mm_body(a_ref, b_ref, o_ref):
    o_ref[...] = jnp.dot(a_ref[...], b_ref[...],
                         preferred_element_type=jnp.float32)


def _mm(a, b, bm=512):
    m, k = a.shape
    _, n = b.shape
    if m % bm != 0:
        bm = m
    grid = (m // bm,)
    return pl.pallas_call(
        _mm_body,
        grid=grid,
        in_specs=[pl.BlockSpec((bm, k), lambda i: (i, 0)),
                  pl.BlockSpec((k, n), lambda i: (0, 0))],
        out_specs=pl.BlockSpec((bm, n), lambda i: (i, 0)),
        out_shape=jax.ShapeDtypeStruct((m, n), jnp.float32),
    )(a, b)


def _mlp_body(hf_ref, w1_ref, b1_ref, w2_ref, b2_ref, w3_ref, b3_ref, o_ref):
    z = jnp.dot(hf_ref[...], w1_ref[...], preferred_element_type=jnp.float32)
    z = jax.nn.relu(z + b1_ref[...])
    z = jnp.dot(z, w2_ref[...], preferred_element_type=jnp.float32)
    z = jax.nn.relu(z + b2_ref[...])
    z = jnp.dot(z, w3_ref[...], preferred_element_type=jnp.float32)
    o_ref[...] = jax.nn.sigmoid(z + b3_ref[...])


def _mlp_head(hf, lw1, lb1, lw2, lb2, lw3, lb3):
    bs = hf.shape[0]
    return pl.pallas_call(
        _mlp_body,
        in_specs=[pl.BlockSpec(hf.shape, lambda: (0, 0)),
                  pl.BlockSpec(lw1.shape, lambda: (0, 0)),
                  pl.BlockSpec((1, 256), lambda: (0, 0)),
                  pl.BlockSpec(lw2.shape, lambda: (0, 0)),
                  pl.BlockSpec((1, 64), lambda: (0, 0)),
                  pl.BlockSpec(lw3.shape, lambda: (0, 0)),
                  pl.BlockSpec((1, 1), lambda: (0, 0))],
        out_specs=pl.BlockSpec((bs, 1), lambda: (0, 0)),
        out_shape=jax.ShapeDtypeStruct((bs, 1), jnp.float32),
    )(hf, lw1, lb1.reshape(1, -1), lw2, lb2.reshape(1, -1), lw3,
      lb3.reshape(1, -1))


def _gat_layer(x, src, dst, W, a_src, a_dst, b, heads, out_ch, concat):
    n = x.shape[0]
    h = _mm(x, W).reshape(n, heads, out_ch)
    alpha_s = (h * a_src[None, :, :]).sum(-1)
    alpha_d = (h * a_dst[None, :, :]).sum(-1)
    alpha = alpha_s[src] + alpha_d[dst]
    alpha = jax.nn.leaky_relu(alpha, negative_slope=0.2)
    amax = jax.ops.segment_max(alpha, dst, num_segments=n)
    amax = jnp.where(jnp.isfinite(amax), amax, 0.0)
    ex = jnp.exp(alpha - amax[dst])
    denom = jax.ops.segment_sum(ex, dst, num_segments=n)
    att = ex / (denom[dst] + 1e-16)
    msg = h[src] * att[:, :, None]
    out = jax.ops.segment_sum(msg, dst, num_segments=n)
    if concat:
        out = out.reshape(n, heads * out_ch)
    else:
        out = out.mean(axis=1)
    return out + b


def kernel(x, edge_index, batch, W1, a_src1, a_dst1, b1, W2, a_src2, a_dst2,
           b2, lw1, lb1, lw2, lb2, lw3, lb3):
    n = x.shape[0]
    loop = jnp.arange(n, dtype=edge_index.dtype)
    src = jnp.concatenate([edge_index[0], loop])
    dst = jnp.concatenate([edge_index[1], loop])
    h = jax.nn.relu(_gat_layer(x, src, dst, W1, a_src1, a_dst1, b1, 4, 64, True))
    h = jax.nn.relu(_gat_layer(h, src, dst, W2, a_src2, a_dst2, b2, 4, 32, False))
    bs = n // 33
    hf = h.reshape(bs, 33 * 32)
    return _mlp_head(hf, lw1, lb1, lw2, lb2, lw3, lb3)


# speedup vs baseline: 25.2255x; 25.2255x over previous
import functools

import jax
import jax.numpy as jnp
from jax import lax
from jax.experimental import pallas as pl
from jax.experimental.pallas import tpu as pltpu
from jax.experimental.pallas import tpu_sc as plsc

N = 16896
H = 4
EA = 287232
NT = 32
KC = 128
EPT = 8960
NFULL = 70
TAIL_BASE = NT * EPT
FC = 32


def _mesh():
    return plsc.VectorSubcoreMesh(core_axis_name="c", subcore_axis_name="s",
                                  num_cores=2, num_subcores=16)


def _edge_loop(chunk_fn, wid):
    @pl.loop(0, NFULL)
    def _(i):
        chunk_fn(wid * EPT + i * KC)

    @pl.when(wid < 4)
    def _():
        chunk_fn(TAIL_BASE + wid * KC)


def _p1_body(src_h, dst_h, ast_h, adt_h,
             gidx_h, didx_h, den_h,
             sidx, dix, asr1, adr1, wr1, giv, div, sbuf,
             as_sh, ad_sh, den_sh, sem):
    core = lax.axis_index("c")
    sub = lax.axis_index("s")
    wid = sub * 2 + core
    nrow = N // 16
    sl = pl.ds(sub * nrow, nrow)

    @pl.loop(0, nrow // 16)
    def _(g):
        sbuf[pl.ds(g * 16, 16)] = jnp.full((16,), 0.0, jnp.float32)

    for k in range(H):
        pltpu.sync_copy(sbuf, den_sh[k].at[sl])
    for k in range(H):
        pltpu.sync_copy(ast_h.at[pl.ds(k * N + sub * nrow, nrow)], sbuf)
        pltpu.sync_copy(sbuf, as_sh[k].at[sl])
        pltpu.sync_copy(adt_h.at[pl.ds(k * N + sub * nrow, nrow)], sbuf)
        pltpu.sync_copy(sbuf, ad_sh[k].at[sl])
    plsc.subcore_barrier()

    def chunk(base):
        pltpu.sync_copy(src_h.at[pl.ds(base, KC)], sidx)
        pltpu.sync_copy(dst_h.at[pl.ds(base, KC)], dix)
        for k in range(H):
            cp_a = pltpu.async_copy(as_sh[k].at[sidx], asr1, sem)
            cp_b = pltpu.async_copy(ad_sh[k].at[dix], adr1, sem)
            cp_a.wait()
            cp_b.wait()

            @pl.loop(0, KC // 16)
            def _(g, k=k):
                t = asr1[pl.ds(g * 16, 16)] + adr1[pl.ds(g * 16, 16)]
                w = jnp.exp(jnp.maximum(t, 0.2 * t))
                wr1[pl.ds(g * 16, 16)] = w
                zi = lax.iota(jnp.int32, 16) * 0
                off = jnp.where(t <= 0.0, zi + N, zi)
                s16 = sidx[pl.ds(g * 16, 16)]
                d16 = dix[pl.ds(g * 16, 16)]
                giv[k][pl.ds(g * 16, 16)] = s16 + off
                div[k][pl.ds(g * 16, 16)] = d16 + off

            pltpu.sync_copy(wr1, den_sh[k].at[dix], add=True)
            pltpu.sync_copy(giv[k], gidx_h.at[pl.ds(k * EA + base, KC)])
            pltpu.sync_copy(div[k], didx_h.at[pl.ds(k * EA + base, KC)])

    _edge_loop(chunk, wid)
    plsc.subcore_barrier()
    for k in range(H):
        pltpu.sync_copy(den_sh[k].at[sl], sbuf)
        pltpu.sync_copy(sbuf,
                        den_h.at[pl.ds((core * H + k) * N + sub * nrow, nrow)])


def _p1_call(src, dst, ast, adt):
    f = pl.kernel(
        _p1_body,
        out_type=[jax.ShapeDtypeStruct((H * EA,), jnp.int32),
                  jax.ShapeDtypeStruct((H * EA,), jnp.int32),
                  jax.ShapeDtypeStruct((2 * H * N,), jnp.float32)],
        mesh=_mesh(),
        scratch_types=[pltpu.VMEM((KC,), jnp.int32),
                       pltpu.VMEM((KC,), jnp.int32),
                       pltpu.VMEM((KC,), jnp.float32),
                       pltpu.VMEM((KC,), jnp.float32),
                       pltpu.VMEM((KC,), jnp.float32),
                       [pltpu.VMEM((KC,), jnp.int32)] * H,
                       [pltpu.VMEM((KC,), jnp.int32)] * H,
                       pltpu.VMEM((N // 16,), jnp.float32),
                       [pltpu.VMEM_SHARED((N,), jnp.float32)] * H,
                       [pltpu.VMEM_SHARED((N,), jnp.float32)] * H,
                       [pltpu.VMEM_SHARED((N,), jnp.float32)] * H,
                       pltpu.SemaphoreType.DMA],
    )
    return f(src, dst, ast, adt)


def _p2_body(nch, gidx_h, didx_h, *rest):
    np_ = nch * H
    tables = rest[:np_]
    parts = rest[np_:2 * np_]
    gi, di, rows, vbuf, acc_sh, sem = rest[2 * np_:]
    core = lax.axis_index("c")
    sub = lax.axis_index("s")
    wid = sub * 2 + core
    mrow = 2 * N // 16
    vrow = mrow // 4
    for p in range(np_):
        k = p // nch

        @pl.loop(0, vrow)
        def _(r):
            vbuf[r, pl.ds(0, 16)] = jnp.full((16,), 0.0, jnp.float32)
            vbuf[r, pl.ds(16, 16)] = jnp.full((16,), 0.0, jnp.float32)

        for j in range(4):
            pltpu.sync_copy(vbuf,
                            acc_sh.at[pl.ds(sub * mrow + j * vrow, vrow)])
        plsc.subcore_barrier()

        def chunk(base, k=k, p=p):
            pltpu.sync_copy(gidx_h.at[pl.ds(k * EA + base, KC)], gi)
            pltpu.sync_copy(didx_h.at[pl.ds(k * EA + base, KC)], di)
            pltpu.async_copy(tables[p].at[gi], rows, sem).wait()
            pltpu.sync_copy(rows, acc_sh.at[di], add=True)

        _edge_loop(chunk, wid)
        plsc.subcore_barrier()
        for j in range(4):
            pltpu.sync_copy(acc_sh.at[pl.ds(sub * mrow + j * vrow, vrow)],
                            vbuf)
            pltpu.sync_copy(vbuf,
                            parts[p].at[core,
                                        pl.ds(sub * mrow + j * vrow, vrow)])
        plsc.subcore_barrier()


def _p2_call(nch, gidx, didx, tables):
    np_ = nch * H
    f = pl.kernel(
        functools.partial(_p2_body, nch),
        out_type=[jax.ShapeDtypeStruct((2, 2 * N, FC), jnp.float32)] * np_,
        mesh=_mesh(),
        scratch_types=[pltpu.VMEM((KC,), jnp.int32),
                       pltpu.VMEM((KC,), jnp.int32),
                       pltpu.VMEM((KC, FC), jnp.float32),
                       pltpu.VMEM((2 * N // 64, FC), jnp.float32),
                       pltpu.VMEM_SHARED((2 * N, FC), jnp.float32),
                       pltpu.SemaphoreType.DMA],
        compiler_params=pltpu.CompilerParams(use_tc_tiling_on_sc=False),
    )
    return f(gidx, didx, *tables)


def _prep_body(nch, f, x_ref, w_ref, as_ref, ad_ref, *outs):
    np_ = nch * H
    tbl = outs[:np_]
    aso_ref, ado_ref = outs[np_], outs[np_ + 1]
    h = jnp.dot(x_ref[...], w_ref[...], preferred_element_type=jnp.float32)
    asv = jnp.dot(h, as_ref[...], preferred_element_type=jnp.float32)
    adv = jnp.dot(h, ad_ref[...], preferred_element_type=jnp.float32)
    aso_ref[...] = asv
    ado_ref[...] = adv
    f1 = jnp.exp(asv)
    f2 = jnp.exp(0.2 * asv)
    for p in range(np_):
        k, c = p // nch, p % nch
        cols = h[:, k * f + c * FC: k * f + (c + 1) * FC]
        tbl[p][0] = f1[:, k:k + 1] * cols
        tbl[p][1] = f2[:, k:k + 1] * cols


def _prep_call(nch, f, x, W, asm, adm):
    np_ = nch * H
    cin = x.shape[1]
    c = W.shape[1]
    bm = 512
    grid = (N // bm,)
    return pl.pallas_call(
        functools.partial(_prep_body, nch, f),
        grid=grid,
        in_specs=[pl.BlockSpec((bm, cin), lambda i: (i, 0)),
                  pl.BlockSpec((cin, c), lambda i: (0, 0)),
                  pl.BlockSpec((c, H), lambda i: (0, 0)),
                  pl.BlockSpec((c, H), lambda i: (0, 0))],
        out_specs=[pl.BlockSpec((2, bm, FC), lambda i: (0, i, 0))] * np_
                  + [pl.BlockSpec((bm, H), lambda i: (i, 0))] * 2,
        out_shape=[jax.ShapeDtypeStruct((2, N, FC), jnp.float32)] * np_
                  + [jax.ShapeDtypeStruct((N, H), jnp.float32)] * 2,
    )(x, W, asm, adm)


def _combine_body(nch, concat, ad_ref, den_ref, b_ref, *rest):
    np_ = nch * H
    parts = rest[:np_]
    out_ref = rest[np_]
    dent = den_ref[0] + den_ref[1]
    f1 = jnp.exp(ad_ref[...])
    f2 = jnp.exp(0.2 * ad_ref[...])
    pieces = []
    for p in range(np_):
        k = p // nch
        pr = parts[p]
        acc1 = pr[0, 0] + pr[1, 0]
        acc2 = pr[0, 1] + pr[1, 1]
        num = f1[:, k:k + 1] * acc1 + f2[:, k:k + 1] * acc2
        pieces.append(num / (dent[:, k:k + 1] + 1e-16))
    if concat:
        o = jnp.concatenate(pieces, axis=1)
    else:
        o = sum(pieces) / float(np_)
    out_ref[...] = jax.nn.relu(o + b_ref[...])


def _combine_call(nch, concat, adv, den, b, parts):
    np_ = nch * H
    cout = b.shape[0]
    bm = 512
    return pl.pallas_call(
        functools.partial(_combine_body, nch, concat),
        grid=(N // bm,),
        in_specs=[pl.BlockSpec((bm, H), lambda i: (i, 0)),
                  pl.BlockSpec((2, bm, H), lambda i: (0, i, 0)),
                  pl.BlockSpec((1, cout), lambda i: (0, 0))]
                 + [pl.BlockSpec((2, 2, bm, FC), lambda i: (0, 0, i, 0))] * np_,
        out_specs=pl.BlockSpec((bm, cout), lambda i: (i, 0)),
        out_shape=jax.ShapeDtypeStruct((N, cout), jnp.float32),
    )(adv, den, b.reshape(1, -1), *parts)


def _mlp_body(hf_ref, w1_ref, b1_ref, w2_ref, b2_ref, w3_ref, b3_ref, o_ref):
    z = jnp.dot(hf_ref[...], w1_ref[...], preferred_element_type=jnp.float32)
    z = jax.nn.relu(z + b1_ref[...])
    z = jnp.dot(z, w2_ref[...], preferred_element_type=jnp.float32)
    z = jax.nn.relu(z + b2_ref[...])
    z = jnp.dot(z, w3_ref[...], preferred_element_type=jnp.float32)
    o_ref[...] = jax.nn.sigmoid(z + b3_ref[...])


def _mlp_head(hf, lw1, lb1, lw2, lb2, lw3, lb3):
    bs = hf.shape[0]
    return pl.pallas_call(
        _mlp_body,
        in_specs=[pl.BlockSpec(hf.shape, lambda: (0, 0)),
                  pl.BlockSpec(lw1.shape, lambda: (0, 0)),
                  pl.BlockSpec((1, 256), lambda: (0, 0)),
                  pl.BlockSpec(lw2.shape, lambda: (0, 0)),
                  pl.BlockSpec((1, 64), lambda: (0, 0)),
                  pl.BlockSpec(lw3.shape, lambda: (0, 0)),
                  pl.BlockSpec((1, 1), lambda: (0, 0))],
        out_specs=pl.BlockSpec((bs, 1), lambda: (0, 0)),
        out_shape=jax.ShapeDtypeStruct((bs, 1), jnp.float32),
    )(hf, lw1, lb1.reshape(1, -1), lw2, lb2.reshape(1, -1), lw3,
      lb3.reshape(1, -1))


def _diag_mix(a):
    h, f = a.shape
    eye = jnp.eye(h, dtype=a.dtype)
    return (a[:, :, None] * eye[:, None, :]).reshape(h * f, h)


def _gat_layer(x, src, dst, W, a_src, a_dst, b, f, nch, concat):
    outs = _prep_call(nch, f, x, W, _diag_mix(a_src), _diag_mix(a_dst))
    np_ = nch * H
    tables = [t.reshape(2 * N, FC) for t in outs[:np_]]
    asv, adv = outs[np_], outs[np_ + 1]
    gidx, didx, den = _p1_call(src, dst, asv.T.reshape(-1),
                               adv.T.reshape(-1))
    parts = _p2_call(nch, gidx, didx, tables)
    parts = [p.reshape(2, 2, N, FC) for p in parts]
    den_t = den.reshape(2, H, N).transpose(0, 2, 1)
    return _combine_call(nch, concat, adv, den_t, b, parts)


def kernel(x, edge_index, batch, W1, a_src1, a_dst1, b1, W2, a_src2, a_dst2,
           b2, lw1, lb1, lw2, lb2, lw3, lb3):
    loop = jnp.arange(N, dtype=edge_index.dtype)
    src = jnp.concatenate([edge_index[0], loop])
    dst = jnp.concatenate([edge_index[1], loop])
    h = _gat_layer(x, src, dst, W1, a_src1, a_dst1, b1, 64, 2, True)
    h = _gat_layer(h, src, dst, W2, a_src2, a_dst2, b2, 32, 1, False)
    hf = h.reshape(N // 33, 33 * 32)
    return _mlp_head(hf, lw1, lb1, lw2, lb2, lw3, lb3)


# speedup vs baseline: 45.1044x; 1.7880x over previous
import functools

import jax
import jax.numpy as jnp
from jax import lax
from jax.experimental import pallas as pl
from jax.experimental.pallas import tpu as pltpu
from jax.experimental.pallas import tpu_sc as plsc

N = 16896
H = 4
EA = 287232
NT = 32
KC = 128
EPT = 8960
NFULL = 70
TAIL_BASE = NT * EPT
FC = 32


def _mesh():
    return plsc.VectorSubcoreMesh(core_axis_name="c", subcore_axis_name="s",
                                  num_cores=2, num_subcores=16)


def _edge_loop(chunk_fn, wid):
    @pl.loop(0, NFULL)
    def _(i):
        chunk_fn(wid * EPT + i * KC)

    @pl.when(wid < 4)
    def _():
        chunk_fn(TAIL_BASE + wid * KC)


def _p1_body(src_h, dst_h, ast_h, adt_h,
             gidx_h, didx_h, den_h,
             sidx, dix, asr1, adr1, wr1, giv, div, sbuf,
             as_sh, ad_sh, den_sh, sem):
    core = lax.axis_index("c")
    sub = lax.axis_index("s")
    wid = sub * 2 + core
    nrow = N // 16
    sl = pl.ds(sub * nrow, nrow)

    @pl.loop(0, nrow // 16)
    def _(g):
        sbuf[pl.ds(g * 16, 16)] = jnp.full((16,), 0.0, jnp.float32)

    for k in range(H):
        pltpu.sync_copy(sbuf, den_sh[k].at[sl])
    for k in range(H):
        pltpu.sync_copy(ast_h.at[pl.ds(k * N + sub * nrow, nrow)], sbuf)
        pltpu.sync_copy(sbuf, as_sh[k].at[sl])
        pltpu.sync_copy(adt_h.at[pl.ds(k * N + sub * nrow, nrow)], sbuf)
        pltpu.sync_copy(sbuf, ad_sh[k].at[sl])
    plsc.subcore_barrier()

    def chunk(base):
        pltpu.sync_copy(src_h.at[pl.ds(base, KC)], sidx)
        pltpu.sync_copy(dst_h.at[pl.ds(base, KC)], dix)
        for k in range(H):
            cp_a = pltpu.async_copy(as_sh[k].at[sidx], asr1, sem)
            cp_b = pltpu.async_copy(ad_sh[k].at[dix], adr1, sem)
            cp_a.wait()
            cp_b.wait()

            @pl.loop(0, KC // 16)
            def _(g, k=k):
                t = asr1[pl.ds(g * 16, 16)] + adr1[pl.ds(g * 16, 16)]
                w = jnp.exp(jnp.maximum(t, 0.2 * t))
                wr1[pl.ds(g * 16, 16)] = w
                zi = lax.iota(jnp.int32, 16) * 0
                off = jnp.where(t <= 0.0, zi + N, zi)
                s16 = sidx[pl.ds(g * 16, 16)]
                d16 = dix[pl.ds(g * 16, 16)]
                giv[k][pl.ds(g * 16, 16)] = s16 + off
                div[k][pl.ds(g * 16, 16)] = d16 + off

            pltpu.sync_copy(wr1, den_sh[k].at[dix], add=True)
            pltpu.sync_copy(giv[k], gidx_h.at[pl.ds(k * EA + base, KC)])
            pltpu.sync_copy(div[k], didx_h.at[pl.ds(k * EA + base, KC)])

    _edge_loop(chunk, wid)
    plsc.subcore_barrier()
    for k in range(H):
        pltpu.sync_copy(den_sh[k].at[sl], sbuf)
        pltpu.sync_copy(sbuf,
                        den_h.at[pl.ds((core * H + k) * N + sub * nrow, nrow)])


def _p1_call(src, dst, ast, adt):
    f = pl.kernel(
        _p1_body,
        out_type=[jax.ShapeDtypeStruct((H * EA,), jnp.int32),
                  jax.ShapeDtypeStruct((H * EA,), jnp.int32),
                  jax.ShapeDtypeStruct((2 * H * N,), jnp.float32)],
        mesh=_mesh(),
        scratch_types=[pltpu.VMEM((KC,), jnp.int32),
                       pltpu.VMEM((KC,), jnp.int32),
                       pltpu.VMEM((KC,), jnp.float32),
                       pltpu.VMEM((KC,), jnp.float32),
                       pltpu.VMEM((KC,), jnp.float32),
                       [pltpu.VMEM((KC,), jnp.int32)] * H,
                       [pltpu.VMEM((KC,), jnp.int32)] * H,
                       pltpu.VMEM((N // 16,), jnp.float32),
                       [pltpu.VMEM_SHARED((N,), jnp.float32)] * H,
                       [pltpu.VMEM_SHARED((N,), jnp.float32)] * H,
                       [pltpu.VMEM_SHARED((N,), jnp.float32)] * H,
                       pltpu.SemaphoreType.DMA],
    )
    return f(src, dst, ast, adt)


EPT2 = 17920
NFULL2 = 140


def _p2_body(nch, gidx_h, didx_h, *rest):
    np_ = nch * H
    np2 = np_ // 2
    tables = rest[:np_]
    parts = rest[np_:2 * np_]
    gi2, di2, rows2, vbuf, acc_sh, sem_i, sem_g = rest[2 * np_:]
    core = lax.axis_index("c")
    sub = lax.axis_index("s")
    mrow = 2 * N // 16
    vrow = mrow // 4

    def sweep(q):
        kk = q // nch
        hb = kk * EA + sub * EPT2

        def fire_idx(i, b):
            base = hb + i * KC
            pltpu.async_copy(gidx_h.at[pl.ds(base, KC)], gi2.at[b], sem_i[b])
            pltpu.async_copy(didx_h.at[pl.ds(base, KC)], di2.at[b], sem_i[b])

        def wait_idx(b):
            pltpu.make_async_copy(gidx_h.at[pl.ds(0, KC)], gi2.at[b],
                                  sem_i[b]).wait()
            pltpu.make_async_copy(didx_h.at[pl.ds(0, KC)], di2.at[b],
                                  sem_i[b]).wait()

        def fire_gather(b):
            wait_idx(b)
            pltpu.async_copy(tables[q].at[gi2.at[b]], rows2.at[b], sem_g[b])

        def drain(b):
            pltpu.make_async_copy(tables[q].at[pl.ds(0, KC)], rows2.at[b],
                                  sem_g[b]).wait()
            pltpu.sync_copy(rows2.at[b], acc_sh.at[di2.at[b]], add=True)

        fire_idx(0, 0)
        fire_idx(1, 1)

        @pl.loop(0, NFULL2 // 2)
        def _(m):
            fire_gather(0)
            fire_gather(1)
            drain(0)
            fire_idx(2 * m + 2, 0)
            drain(1)
            fire_idx(2 * m + 3, 1)

        wait_idx(0)
        wait_idx(1)

        @pl.when(sub < 4)
        def _():
            base = kk * EA + TAIL_BASE + sub * KC
            pltpu.async_copy(gidx_h.at[pl.ds(base, KC)], gi2.at[0], sem_i[0])
            pltpu.async_copy(didx_h.at[pl.ds(base, KC)], di2.at[0], sem_i[0])
            fire_gather(0)
            drain(0)

    for p in range(np2):
        @pl.loop(0, vrow)
        def _(r):
            vbuf[r, pl.ds(0, 16)] = jnp.full((16,), 0.0, jnp.float32)
            vbuf[r, pl.ds(16, 16)] = jnp.full((16,), 0.0, jnp.float32)

        for j in range(4):
            pltpu.sync_copy(vbuf,
                            acc_sh.at[pl.ds(sub * mrow + j * vrow, vrow)])
        plsc.subcore_barrier()
        for c in range(2):
            @pl.when(core == c)
            def _(c=c, p=p):
                sweep(c * np2 + p)
        plsc.subcore_barrier()
        for c in range(2):
            @pl.when(core == c)
            def _(c=c, p=p):
                q = c * np2 + p
                for j in range(4):
                    sl = pl.ds(sub * mrow + j * vrow, vrow)
                    pltpu.sync_copy(acc_sh.at[sl], vbuf)
                    pltpu.sync_copy(vbuf, parts[q].at[sl])
        plsc.subcore_barrier()


def _p2_call(nch, gidx, didx, tables):
    np_ = nch * H
    f = pl.kernel(
        functools.partial(_p2_body, nch),
        out_type=[jax.ShapeDtypeStruct((2 * N, FC), jnp.float32)] * np_,
        mesh=_mesh(),
        scratch_types=[pltpu.VMEM((2, KC), jnp.int32),
                       pltpu.VMEM((2, KC), jnp.int32),
                       pltpu.VMEM((2, KC, FC), jnp.float32),
                       pltpu.VMEM((2 * N // 64, FC), jnp.float32),
                       pltpu.VMEM_SHARED((2 * N, FC), jnp.float32),
                       [pltpu.SemaphoreType.DMA] * 2,
                       [pltpu.SemaphoreType.DMA] * 2],
        compiler_params=pltpu.CompilerParams(use_tc_tiling_on_sc=False),
    )
    return f(gidx, didx, *tables)


def _prep_body(nch, f, x_ref, w_ref, as_ref, ad_ref, *outs):
    np_ = nch * H
    tbl = outs[:np_]
    aso_ref, ado_ref = outs[np_], outs[np_ + 1]
    h = jnp.dot(x_ref[...], w_ref[...], preferred_element_type=jnp.float32)
    asv = jnp.dot(h, as_ref[...], preferred_element_type=jnp.float32)
    adv = jnp.dot(h, ad_ref[...], preferred_element_type=jnp.float32)
    aso_ref[...] = asv
    ado_ref[...] = adv
    f1 = jnp.exp(asv)
    f2 = jnp.exp(0.2 * asv)
    for p in range(np_):
        k, c = p // nch, p % nch
        cols = h[:, k * f + c * FC: k * f + (c + 1) * FC]
        tbl[p][0] = f1[:, k:k + 1] * cols
        tbl[p][1] = f2[:, k:k + 1] * cols


def _prep_call(nch, f, x, W, asm, adm):
    np_ = nch * H
    cin = x.shape[1]
    c = W.shape[1]
    bm = 512
    grid = (N // bm,)
    return pl.pallas_call(
        functools.partial(_prep_body, nch, f),
        grid=grid,
        in_specs=[pl.BlockSpec((bm, cin), lambda i: (i, 0)),
                  pl.BlockSpec((cin, c), lambda i: (0, 0)),
                  pl.BlockSpec((c, H), lambda i: (0, 0)),
                  pl.BlockSpec((c, H), lambda i: (0, 0))],
        out_specs=[pl.BlockSpec((2, bm, FC), lambda i: (0, i, 0))] * np_
                  + [pl.BlockSpec((bm, H), lambda i: (i, 0))] * 2,
        out_shape=[jax.ShapeDtypeStruct((2, N, FC), jnp.float32)] * np_
                  + [jax.ShapeDtypeStruct((N, H), jnp.float32)] * 2,
    )(x, W, asm, adm)


def _combine_body(nch, concat, ad_ref, den_ref, b_ref, *rest):
    np_ = nch * H
    parts = rest[:np_]
    out_ref = rest[np_]
    dent = den_ref[0] + den_ref[1]
    f1 = jnp.exp(ad_ref[...])
    f2 = jnp.exp(0.2 * ad_ref[...])
    pieces = []
    for p in range(np_):
        k = p // nch
        pr = parts[p]
        acc1 = pr[0]
        acc2 = pr[1]
        num = f1[:, k:k + 1] * acc1 + f2[:, k:k + 1] * acc2
        pieces.append(num / (dent[:, k:k + 1] + 1e-16))
    if concat:
        o = jnp.concatenate(pieces, axis=1)
    else:
        o = sum(pieces) / float(np_)
    out_ref[...] = jax.nn.relu(o + b_ref[...])


def _combine_call(nch, concat, adv, den, b, parts):
    np_ = nch * H
    cout = b.shape[0]
    bm = 512
    return pl.pallas_call(
        functools.partial(_combine_body, nch, concat),
        grid=(N // bm,),
        in_specs=[pl.BlockSpec((bm, H), lambda i: (i, 0)),
                  pl.BlockSpec((2, bm, H), lambda i: (0, i, 0)),
                  pl.BlockSpec((1, cout), lambda i: (0, 0))]
                 + [pl.BlockSpec((2, bm, FC), lambda i: (0, i, 0))] * np_,
        out_specs=pl.BlockSpec((bm, cout), lambda i: (i, 0)),
        out_shape=jax.ShapeDtypeStruct((N, cout), jnp.float32),
    )(adv, den, b.reshape(1, -1), *parts)


def _mlp_body(hf_ref, w1_ref, b1_ref, w2_ref, b2_ref, w3_ref, b3_ref, o_ref):
    z = jnp.dot(hf_ref[...], w1_ref[...], preferred_element_type=jnp.float32)
    z = jax.nn.relu(z + b1_ref[...])
    z = jnp.dot(z, w2_ref[...], preferred_element_type=jnp.float32)
    z = jax.nn.relu(z + b2_ref[...])
    z = jnp.dot(z, w3_ref[...], preferred_element_type=jnp.float32)
    o_ref[...] = jax.nn.sigmoid(z + b3_ref[...])


def _mlp_head(hf, lw1, lb1, lw2, lb2, lw3, lb3):
    bs = hf.shape[0]
    return pl.pallas_call(
        _mlp_body,
        in_specs=[pl.BlockSpec(hf.shape, lambda: (0, 0)),
                  pl.BlockSpec(lw1.shape, lambda: (0, 0)),
                  pl.BlockSpec((1, 256), lambda: (0, 0)),
                  pl.BlockSpec(lw2.shape, lambda: (0, 0)),
                  pl.BlockSpec((1, 64), lambda: (0, 0)),
                  pl.BlockSpec(lw3.shape, lambda: (0, 0)),
                  pl.BlockSpec((1, 1), lambda: (0, 0))],
        out_specs=pl.BlockSpec((bs, 1), lambda: (0, 0)),
        out_shape=jax.ShapeDtypeStruct((bs, 1), jnp.float32),
    )(hf, lw1, lb1.reshape(1, -1), lw2, lb2.reshape(1, -1), lw3,
      lb3.reshape(1, -1))


def _diag_mix(a):
    h, f = a.shape
    eye = jnp.eye(h, dtype=a.dtype)
    return (a[:, :, None] * eye[:, None, :]).reshape(h * f, h)


def _gat_layer(x, src, dst, W, a_src, a_dst, b, f, nch, concat):
    outs = _prep_call(nch, f, x, W, _diag_mix(a_src), _diag_mix(a_dst))
    np_ = nch * H
    tables = [t.reshape(2 * N, FC) for t in outs[:np_]]
    asv, adv = outs[np_], outs[np_ + 1]
    gidx, didx, den = _p1_call(src, dst, asv.T.reshape(-1),
                               adv.T.reshape(-1))
    parts = _p2_call(nch, gidx, didx, tables)
    parts = [p.reshape(2, N, FC) for p in parts]
    den_t = den.reshape(2, H, N).transpose(0, 2, 1)
    return _combine_call(nch, concat, adv, den_t, b, parts)


def kernel(x, edge_index, batch, W1, a_src1, a_dst1, b1, W2, a_src2, a_dst2,
           b2, lw1, lb1, lw2, lb2, lw3, lb3):
    loop = jnp.arange(N, dtype=edge_index.dtype)
    src = jnp.concatenate([edge_index[0], loop])
    dst = jnp.concatenate([edge_index[1], loop])
    h = _gat_layer(x, src, dst, W1, a_src1, a_dst1, b1, 64, 2, True)
    h = _gat_layer(h, src, dst, W2, a_src2, a_dst2, b2, 32, 1, False)
    hf = h.reshape(N // 33, 33 * 32)
    return _mlp_head(hf, lw1, lb1, lw2, lb2, lw3, lb3)


# speedup vs baseline: 51.9285x; 1.1513x over previous
import functools

import jax
import jax.numpy as jnp
from jax import lax
from jax.experimental import pallas as pl
from jax.experimental.pallas import tpu as pltpu
from jax.experimental.pallas import tpu_sc as plsc

N = 16896
H = 4
EA = 287232
NT = 32
KC = 128
EPT = 8960
NFULL = 70
TAIL_BASE = NT * EPT
FC = 32


def _mesh():
    return plsc.VectorSubcoreMesh(core_axis_name="c", subcore_axis_name="s",
                                  num_cores=2, num_subcores=16)


def _edge_loop(chunk_fn, wid):
    @pl.loop(0, NFULL)
    def _(i):
        chunk_fn(wid * EPT + i * KC)

    @pl.when(wid < 4)
    def _():
        chunk_fn(TAIL_BASE + wid * KC)


def _p1_body(src_h, dst_h, ast_h, adt_h,
             gidx_h, didx_h, den_h,
             sidx2, dix2, asr4, adr4, wr1, giv2, div2, sbuf,
             as_sh, ad_sh, den_sh, sem_i, sem_g, sem_o):
    core = lax.axis_index("c")
    sub = lax.axis_index("s")
    wid = sub * 2 + core
    nrow = N // 16
    sl = pl.ds(sub * nrow, nrow)

    @pl.loop(0, nrow // 16)
    def _(g):
        sbuf[pl.ds(g * 16, 16)] = jnp.full((16,), 0.0, jnp.float32)

    for k in range(H):
        pltpu.sync_copy(sbuf, den_sh[k].at[sl])
    for k in range(H):
        pltpu.sync_copy(ast_h.at[pl.ds(k * N + sub * nrow, nrow)], sbuf)
        pltpu.sync_copy(sbuf, as_sh[k].at[sl])
        pltpu.sync_copy(adt_h.at[pl.ds(k * N + sub * nrow, nrow)], sbuf)
        pltpu.sync_copy(sbuf, ad_sh[k].at[sl])
    plsc.subcore_barrier()

    hbase = wid * EPT

    def fire_idx(i, b):
        base = hbase + i * KC
        pltpu.async_copy(src_h.at[pl.ds(base, KC)], sidx2.at[b], sem_i[b])
        pltpu.async_copy(dst_h.at[pl.ds(base, KC)], dix2.at[b], sem_i[b])

    def wait_idx(b):
        pltpu.make_async_copy(src_h.at[pl.ds(0, KC)], sidx2.at[b],
                              sem_i[b]).wait()
        pltpu.make_async_copy(dst_h.at[pl.ds(0, KC)], dix2.at[b],
                              sem_i[b]).wait()

    def fire_gathers(b):
        for k in range(H):
            pltpu.async_copy(as_sh[k].at[sidx2.at[b]], asr4.at[b, k],
                             sem_g[b])
            pltpu.async_copy(ad_sh[k].at[dix2.at[b]], adr4.at[b, k],
                             sem_g[b])

    def wait_gathers(b):
        for k in range(H):
            pltpu.make_async_copy(ast_h.at[pl.ds(0, KC)], asr4.at[b, k],
                                  sem_g[b]).wait()
            pltpu.make_async_copy(ast_h.at[pl.ds(0, KC)], adr4.at[b, k],
                                  sem_g[b]).wait()

    def drain_out(b):
        pltpu.make_async_copy(gidx_h.at[pl.ds(0, H * KC)], giv2.at[b],
                              sem_o[b]).wait()
        pltpu.make_async_copy(gidx_h.at[pl.ds(0, H * KC)], div2.at[b],
                              sem_o[b]).wait()

    def compute(b):
        for k in range(H):
            @pl.loop(0, KC // 16)
            def _(g, k=k, b=b):
                t = asr4[b, k, pl.ds(g * 16, 16)] \
                    + adr4[b, k, pl.ds(g * 16, 16)]
                w = jnp.exp(jnp.maximum(t, 0.2 * t))
                wr1[pl.ds(g * 16, 16)] = w
                zi = lax.iota(jnp.int32, 16) * 0
                off = jnp.where(t <= 0.0, zi + N, zi)
                s16 = sidx2[b, pl.ds(g * 16, 16)]
                d16 = dix2[b, pl.ds(g * 16, 16)]
                giv2[b, pl.ds(k * KC + g * 16, 16)] = s16 + off
                div2[b, pl.ds(k * KC + g * 16, 16)] = d16 + off

            pltpu.sync_copy(wr1, den_sh[k].at[dix2.at[b]], add=True)

    def fire_out(ci, b):
        pltpu.async_copy(giv2.at[b], gidx_h.at[pl.ds(ci * H * KC, H * KC)],
                         sem_o[b])
        pltpu.async_copy(div2.at[b], didx_h.at[pl.ds(ci * H * KC, H * KC)],
                         sem_o[b])

    fire_idx(0, 0)
    fire_idx(1, 1)
    wait_idx(0)
    fire_gathers(0)

    @pl.loop(0, NFULL // 2)
    def _(m):
        for b in range(2):
            i = 2 * m + b
            nb = 1 - b
            wait_idx(nb)
            fire_gathers(nb)
            wait_gathers(b)

            @pl.when(m > 0)
            def _(b=b):
                drain_out(b)

            compute(b)
            fire_out(wid * NFULL + i, b)
            fire_idx(i + 2, b)

    wait_gathers(0)
    wait_idx(1)
    drain_out(0)
    drain_out(1)

    @pl.when(wid < 4)
    def _():
        base = TAIL_BASE + wid * KC
        pltpu.sync_copy(src_h.at[pl.ds(base, KC)], sidx2.at[0])
        pltpu.sync_copy(dst_h.at[pl.ds(base, KC)], dix2.at[0])
        fire_gathers(0)
        wait_gathers(0)
        compute(0)
        fire_out(TAIL_BASE // KC + wid, 0)
        drain_out(0)

    plsc.subcore_barrier()
    for k in range(H):
        pltpu.sync_copy(den_sh[k].at[sl], sbuf)
        pltpu.sync_copy(sbuf,
                        den_h.at[pl.ds((core * H + k) * N + sub * nrow, nrow)])


def _p1_call(src, dst, ast, adt):
    f = pl.kernel(
        _p1_body,
        out_type=[jax.ShapeDtypeStruct((H * EA,), jnp.int32),
                  jax.ShapeDtypeStruct((H * EA,), jnp.int32),
                  jax.ShapeDtypeStruct((2 * H * N,), jnp.float32)],
        mesh=_mesh(),
        scratch_types=[pltpu.VMEM((2, KC), jnp.int32),
                       pltpu.VMEM((2, KC), jnp.int32),
                       pltpu.VMEM((2, H, KC), jnp.float32),
                       pltpu.VMEM((2, H, KC), jnp.float32),
                       pltpu.VMEM((KC,), jnp.float32),
                       pltpu.VMEM((2, H * KC), jnp.int32),
                       pltpu.VMEM((2, H * KC), jnp.int32),
                       pltpu.VMEM((N // 16,), jnp.float32),
                       [pltpu.VMEM_SHARED((N,), jnp.float32)] * H,
                       [pltpu.VMEM_SHARED((N,), jnp.float32)] * H,
                       [pltpu.VMEM_SHARED((N,), jnp.float32)] * H,
                       [pltpu.SemaphoreType.DMA] * 2,
                       [pltpu.SemaphoreType.DMA] * 2,
                       [pltpu.SemaphoreType.DMA] * 2],
    )
    return f(src, dst, ast, adt)


EPT2 = 17920
NFULL2 = 140


def _p2_body(nch, gidx_h, didx_h, *rest):
    np_ = nch * H
    np2 = np_ // 2
    tables = rest[:np_]
    parts = rest[np_:2 * np_]
    gi2, di2, rows2, vbuf, acc_sh, sem_i, sem_g = rest[2 * np_:]
    core = lax.axis_index("c")
    sub = lax.axis_index("s")
    mrow = 2 * N // 16
    vrow = mrow // 4

    def sweep(q):
        kk = q // nch
        cbase = sub * NFULL2

        def fire_idx(i, b):
            base = ((cbase + i) * H + kk) * KC
            pltpu.async_copy(gidx_h.at[pl.ds(base, KC)], gi2.at[b], sem_i[b])
            pltpu.async_copy(didx_h.at[pl.ds(base, KC)], di2.at[b], sem_i[b])

        def wait_idx(b):
            pltpu.make_async_copy(gidx_h.at[pl.ds(0, KC)], gi2.at[b],
                                  sem_i[b]).wait()
            pltpu.make_async_copy(didx_h.at[pl.ds(0, KC)], di2.at[b],
                                  sem_i[b]).wait()

        def fire_gather(b):
            wait_idx(b)
            pltpu.async_copy(tables[q].at[gi2.at[b]], rows2.at[b], sem_g[b])

        def drain(b):
            pltpu.make_async_copy(tables[q].at[pl.ds(0, KC)], rows2.at[b],
                                  sem_g[b]).wait()
            pltpu.sync_copy(rows2.at[b], acc_sh.at[di2.at[b]], add=True)

        fire_idx(0, 0)
        fire_idx(1, 1)

        @pl.loop(0, NFULL2 // 2)
        def _(m):
            fire_gather(0)
            fire_gather(1)
            drain(0)
            fire_idx(2 * m + 2, 0)
            drain(1)
            fire_idx(2 * m + 3, 1)

        wait_idx(0)
        wait_idx(1)

        @pl.when(sub < 4)
        def _():
            base = ((TAIL_BASE // KC + sub) * H + kk) * KC
            pltpu.async_copy(gidx_h.at[pl.ds(base, KC)], gi2.at[0], sem_i[0])
            pltpu.async_copy(didx_h.at[pl.ds(base, KC)], di2.at[0], sem_i[0])
            fire_gather(0)
            drain(0)

    for p in range(np2):
        @pl.loop(0, vrow)
        def _(r):
            vbuf[r, pl.ds(0, 16)] = jnp.full((16,), 0.0, jnp.float32)
            vbuf[r, pl.ds(16, 16)] = jnp.full((16,), 0.0, jnp.float32)

        for j in range(4):
            pltpu.sync_copy(vbuf,
                            acc_sh.at[pl.ds(sub * mrow + j * vrow, vrow)])
        plsc.subcore_barrier()
        for c in range(2):
            @pl.when(core == c)
            def _(c=c, p=p):
                sweep(c * np2 + p)
        plsc.subcore_barrier()
        for c in range(2):
            @pl.when(core == c)
            def _(c=c, p=p):
                q = c * np2 + p
                for j in range(4):
                    sl = pl.ds(sub * mrow + j * vrow, vrow)
                    pltpu.sync_copy(acc_sh.at[sl], vbuf)
                    pltpu.sync_copy(vbuf, parts[q].at[sl])
        plsc.subcore_barrier()


def _p2_call(nch, gidx, didx, tables):
    np_ = nch * H
    f = pl.kernel(
        functools.partial(_p2_body, nch),
        out_type=[jax.ShapeDtypeStruct((2 * N, FC), jnp.float32)] * np_,
        mesh=_mesh(),
        scratch_types=[pltpu.VMEM((2, KC), jnp.int32),
                       pltpu.VMEM((2, KC), jnp.int32),
                       pltpu.VMEM((2, KC, FC), jnp.float32),
                       pltpu.VMEM((2 * N // 64, FC), jnp.float32),
                       pltpu.VMEM_SHARED((2 * N, FC), jnp.float32),
                       [pltpu.SemaphoreType.DMA] * 2,
                       [pltpu.SemaphoreType.DMA] * 2],
        compiler_params=pltpu.CompilerParams(use_tc_tiling_on_sc=False),
    )
    return f(gidx, didx, *tables)


def _prep_body(nch, f, x_ref, w_ref, as_ref, ad_ref, *outs):
    np_ = nch * H
    tbl = outs[:np_]
    aso_ref, ado_ref = outs[np_], outs[np_ + 1]
    h = jnp.dot(x_ref[...], w_ref[...], preferred_element_type=jnp.float32)
    asv = jnp.dot(h, as_ref[...], preferred_element_type=jnp.float32)
    adv = jnp.dot(h, ad_ref[...], preferred_element_type=jnp.float32)
    aso_ref[...] = asv
    ado_ref[...] = adv
    f1 = jnp.exp(asv)
    f2 = jnp.exp(0.2 * asv)
    for p in range(np_):
        k, c = p // nch, p % nch
        cols = h[:, k * f + c * FC: k * f + (c + 1) * FC]
        tbl[p][0] = f1[:, k:k + 1] * cols
        tbl[p][1] = f2[:, k:k + 1] * cols


def _prep_call(nch, f, x, W, asm, adm):
    np_ = nch * H
    cin = x.shape[1]
    c = W.shape[1]
    bm = 512
    grid = (N // bm,)
    return pl.pallas_call(
        functools.partial(_prep_body, nch, f),
        grid=grid,
        in_specs=[pl.BlockSpec((bm, cin), lambda i: (i, 0)),
                  pl.BlockSpec((cin, c), lambda i: (0, 0)),
                  pl.BlockSpec((c, H), lambda i: (0, 0)),
                  pl.BlockSpec((c, H), lambda i: (0, 0))],
        out_specs=[pl.BlockSpec((2, bm, FC), lambda i: (0, i, 0))] * np_
                  + [pl.BlockSpec((bm, H), lambda i: (i, 0))] * 2,
        out_shape=[jax.ShapeDtypeStruct((2, N, FC), jnp.float32)] * np_
                  + [jax.ShapeDtypeStruct((N, H), jnp.float32)] * 2,
    )(x, W, asm, adm)


def _combine_body(nch, concat, ad_ref, den_ref, b_ref, *rest):
    np_ = nch * H
    parts = rest[:np_]
    out_ref = rest[np_]
    dent = den_ref[0] + den_ref[1]
    f1 = jnp.exp(ad_ref[...])
    f2 = jnp.exp(0.2 * ad_ref[...])
    pieces = []
    for p in range(np_):
        k = p // nch
        pr = parts[p]
        acc1 = pr[0]
        acc2 = pr[1]
        num = f1[:, k:k + 1] * acc1 + f2[:, k:k + 1] * acc2
        pieces.append(num / (dent[:, k:k + 1] + 1e-16))
    if concat:
        o = jnp.concatenate(pieces, axis=1)
    else:
        o = sum(pieces) / float(np_)
    out_ref[...] = jax.nn.relu(o + b_ref[...])


def _combine_call(nch, concat, adv, den, b, parts):
    np_ = nch * H
    cout = b.shape[0]
    bm = 512
    return pl.pallas_call(
        functools.partial(_combine_body, nch, concat),
        grid=(N // bm,),
        in_specs=[pl.BlockSpec((bm, H), lambda i: (i, 0)),
                  pl.BlockSpec((2, bm, H), lambda i: (0, i, 0)),
                  pl.BlockSpec((1, cout), lambda i: (0, 0))]
                 + [pl.BlockSpec((2, bm, FC), lambda i: (0, i, 0))] * np_,
        out_specs=pl.BlockSpec((bm, cout), lambda i: (i, 0)),
        out_shape=jax.ShapeDtypeStruct((N, cout), jnp.float32),
    )(adv, den, b.reshape(1, -1), *parts)


def _mlp_body(hf_ref, w1_ref, b1_ref, w2_ref, b2_ref, w3_ref, b3_ref, o_ref):
    z = jnp.dot(hf_ref[...], w1_ref[...], preferred_element_type=jnp.float32)
    z = jax.nn.relu(z + b1_ref[...])
    z = jnp.dot(z, w2_ref[...], preferred_element_type=jnp.float32)
    z = jax.nn.relu(z + b2_ref[...])
    z = jnp.dot(z, w3_ref[...], preferred_element_type=jnp.float32)
    o_ref[...] = jax.nn.sigmoid(z + b3_ref[...])


def _mlp_head(hf, lw1, lb1, lw2, lb2, lw3, lb3):
    bs = hf.shape[0]
    return pl.pallas_call(
        _mlp_body,
        in_specs=[pl.BlockSpec(hf.shape, lambda: (0, 0)),
                  pl.BlockSpec(lw1.shape, lambda: (0, 0)),
                  pl.BlockSpec((1, 256), lambda: (0, 0)),
                  pl.BlockSpec(lw2.shape, lambda: (0, 0)),
                  pl.BlockSpec((1, 64), lambda: (0, 0)),
                  pl.BlockSpec(lw3.shape, lambda: (0, 0)),
                  pl.BlockSpec((1, 1), lambda: (0, 0))],
        out_specs=pl.BlockSpec((bs, 1), lambda: (0, 0)),
        out_shape=jax.ShapeDtypeStruct((bs, 1), jnp.float32),
    )(hf, lw1, lb1.reshape(1, -1), lw2, lb2.reshape(1, -1), lw3,
      lb3.reshape(1, -1))


def _diag_mix(a):
    h, f = a.shape
    eye = jnp.eye(h, dtype=a.dtype)
    return (a[:, :, None] * eye[:, None, :]).reshape(h * f, h)


def _gat_layer(x, src, dst, W, a_src, a_dst, b, f, nch, concat):
    outs = _prep_call(nch, f, x, W, _diag_mix(a_src), _diag_mix(a_dst))
    np_ = nch * H
    tables = [t.reshape(2 * N, FC) for t in outs[:np_]]
    asv, adv = outs[np_], outs[np_ + 1]
    gidx, didx, den = _p1_call(src, dst, asv.T.reshape(-1),
                               adv.T.reshape(-1))
    parts = _p2_call(nch, gidx, didx, tables)
    parts = [p.reshape(2, N, FC) for p in parts]
    den_t = den.reshape(2, H, N).transpose(0, 2, 1)
    return _combine_call(nch, concat, adv, den_t, b, parts)


def kernel(x, edge_index, batch, W1, a_src1, a_dst1, b1, W2, a_src2, a_dst2,
           b2, lw1, lb1, lw2, lb2, lw3, lb3):
    loop = jnp.arange(N, dtype=edge_index.dtype)
    src = jnp.concatenate([edge_index[0], loop])
    dst = jnp.concatenate([edge_index[1], loop])
    h = _gat_layer(x, src, dst, W1, a_src1, a_dst1, b1, 64, 2, True)
    h = _gat_layer(h, src, dst, W2, a_src2, a_dst2, b2, 32, 1, False)
    hf = h.reshape(N // 33, 33 * 32)
    return _mlp_head(hf, lw1, lb1, lw2, lb2, lw3, lb3)


# speedup vs baseline: 52.8067x; 1.0169x over previous
import functools

import jax
import jax.numpy as jnp
from jax import lax
from jax.experimental import pallas as pl
from jax.experimental.pallas import tpu as pltpu
from jax.experimental.pallas import tpu_sc as plsc

N = 16896
H = 4
EA = 287232
NT = 32
KC = 128
EPT = 8960
NFULL = 70
TAIL_BASE = NT * EPT
FC = 32


def _mesh():
    return plsc.VectorSubcoreMesh(core_axis_name="c", subcore_axis_name="s",
                                  num_cores=2, num_subcores=16)


def _edge_loop(chunk_fn, wid):
    @pl.loop(0, NFULL)
    def _(i):
        chunk_fn(wid * EPT + i * KC)

    @pl.when(wid < 4)
    def _():
        chunk_fn(TAIL_BASE + wid * KC)


def _p1_body(src_h, dst_h, ast_h, adt_h,
             gidx_h, didx_h, den_h,
             sidx2, dix2, asr4, adr4, wr1, giv2, div2, sbuf,
             as_sh, ad_sh, den_sh, sem_i, sem_g, sem_o):
    core = lax.axis_index("c")
    sub = lax.axis_index("s")
    wid = sub * 2 + core
    nrow = N // 16
    sl = pl.ds(sub * nrow, nrow)

    @pl.loop(0, nrow // 16)
    def _(g):
        sbuf[pl.ds(g * 16, 16)] = jnp.full((16,), 0.0, jnp.float32)

    for k in range(H):
        pltpu.sync_copy(sbuf, den_sh[k].at[sl])
    for k in range(H):
        pltpu.sync_copy(ast_h.at[pl.ds(k * N + sub * nrow, nrow)], sbuf)
        pltpu.sync_copy(sbuf, as_sh[k].at[sl])
        pltpu.sync_copy(adt_h.at[pl.ds(k * N + sub * nrow, nrow)], sbuf)
        pltpu.sync_copy(sbuf, ad_sh[k].at[sl])
    plsc.subcore_barrier()

    hbase = wid * EPT

    def fire_idx(i, b):
        base = hbase + i * KC
        pltpu.async_copy(src_h.at[pl.ds(base, KC)], sidx2.at[b], sem_i[b])
        pltpu.async_copy(dst_h.at[pl.ds(base, KC)], dix2.at[b], sem_i[b])

    def wait_idx(b):
        pltpu.make_async_copy(src_h.at[pl.ds(0, KC)], sidx2.at[b],
                              sem_i[b]).wait()
        pltpu.make_async_copy(dst_h.at[pl.ds(0, KC)], dix2.at[b],
                              sem_i[b]).wait()

    def fire_gathers(b):
        for k in range(H):
            pltpu.async_copy(as_sh[k].at[sidx2.at[b]], asr4.at[b, k],
                             sem_g[b])
            pltpu.async_copy(ad_sh[k].at[dix2.at[b]], adr4.at[b, k],
                             sem_g[b])

    def wait_gathers(b):
        for k in range(H):
            pltpu.make_async_copy(ast_h.at[pl.ds(0, KC)], asr4.at[b, k],
                                  sem_g[b]).wait()
            pltpu.make_async_copy(ast_h.at[pl.ds(0, KC)], adr4.at[b, k],
                                  sem_g[b]).wait()

    def drain_out(b):
        pltpu.make_async_copy(gidx_h.at[pl.ds(0, H * KC)], giv2.at[b],
                              sem_o[b]).wait()
        pltpu.make_async_copy(gidx_h.at[pl.ds(0, H * KC)], div2.at[b],
                              sem_o[b]).wait()

    def compute(b):
        for k in range(H):
            @pl.loop(0, KC // 16)
            def _(g, k=k, b=b):
                t = asr4[b, k, pl.ds(g * 16, 16)] \
                    + adr4[b, k, pl.ds(g * 16, 16)]
                w = jnp.exp(jnp.maximum(t, 0.2 * t))
                wr1[pl.ds(g * 16, 16)] = w
                zi = lax.iota(jnp.int32, 16) * 0
                off = jnp.where(t <= 0.0, zi + N, zi)
                s16 = sidx2[b, pl.ds(g * 16, 16)]
                d16 = dix2[b, pl.ds(g * 16, 16)]
                giv2[b, pl.ds(k * KC + g * 16, 16)] = s16 + off
                div2[b, pl.ds(k * KC + g * 16, 16)] = d16 + off

            pltpu.sync_copy(wr1, den_sh[k].at[dix2.at[b]], add=True)

    def fire_out(ci, b):
        pltpu.async_copy(giv2.at[b], gidx_h.at[pl.ds(ci * H * KC, H * KC)],
                         sem_o[b])
        pltpu.async_copy(div2.at[b], didx_h.at[pl.ds(ci * H * KC, H * KC)],
                         sem_o[b])

    fire_idx(0, 0)
    fire_idx(1, 1)
    wait_idx(0)
    fire_gathers(0)

    @pl.loop(0, NFULL // 2)
    def _(m):
        for b in range(2):
            i = 2 * m + b
            nb = 1 - b
            wait_idx(nb)
            fire_gathers(nb)
            wait_gathers(b)

            @pl.when(m > 0)
            def _(b=b):
                drain_out(b)

            compute(b)
            fire_out(wid * NFULL + i, b)
            fire_idx(i + 2, b)

    wait_gathers(0)
    wait_idx(1)
    drain_out(0)
    drain_out(1)

    @pl.when(wid < 4)
    def _():
        base = TAIL_BASE + wid * KC
        pltpu.sync_copy(src_h.at[pl.ds(base, KC)], sidx2.at[0])
        pltpu.sync_copy(dst_h.at[pl.ds(base, KC)], dix2.at[0])
        fire_gathers(0)
        wait_gathers(0)
        compute(0)
        fire_out(TAIL_BASE // KC + wid, 0)
        drain_out(0)

    plsc.subcore_barrier()
    for k in range(H):
        pltpu.sync_copy(den_sh[k].at[sl], sbuf)
        pltpu.sync_copy(sbuf,
                        den_h.at[pl.ds((core * H + k) * N + sub * nrow, nrow)])


def _p1_call(src, dst, ast, adt):
    f = pl.kernel(
        _p1_body,
        out_type=[jax.ShapeDtypeStruct((H * EA,), jnp.int32),
                  jax.ShapeDtypeStruct((H * EA,), jnp.int32),
                  jax.ShapeDtypeStruct((2 * H * N,), jnp.float32)],
        mesh=_mesh(),
        scratch_types=[pltpu.VMEM((2, KC), jnp.int32),
                       pltpu.VMEM((2, KC), jnp.int32),
                       pltpu.VMEM((2, H, KC), jnp.float32),
                       pltpu.VMEM((2, H, KC), jnp.float32),
                       pltpu.VMEM((KC,), jnp.float32),
                       pltpu.VMEM((2, H * KC), jnp.int32),
                       pltpu.VMEM((2, H * KC), jnp.int32),
                       pltpu.VMEM((N // 16,), jnp.float32),
                       [pltpu.VMEM_SHARED((N,), jnp.float32)] * H,
                       [pltpu.VMEM_SHARED((N,), jnp.float32)] * H,
                       [pltpu.VMEM_SHARED((N,), jnp.float32)] * H,
                       [pltpu.SemaphoreType.DMA] * 2,
                       [pltpu.SemaphoreType.DMA] * 2,
                       [pltpu.SemaphoreType.DMA] * 2],
    )
    return f(src, dst, ast, adt)


EPT2 = 17920
NFULL2 = 140


def _p2_body(nch, gidx_h, didx_h, *rest):
    np_ = nch * H
    np2 = np_ // 2
    tables = rest[:np_]
    parts = rest[np_:2 * np_]
    gi2, di2, rows2, vbuf, acc_sh, sem_i, sem_g = rest[2 * np_:]
    core = lax.axis_index("c")
    sub = lax.axis_index("s")
    mrow = 2 * N // 16
    vrow = mrow // 4

    def sweep(q):
        kk = q // nch
        cbase = sub * NFULL2

        def fire_idx(i, b):
            base = ((cbase + i) * H + kk) * KC
            pltpu.async_copy(gidx_h.at[pl.ds(base, KC)], gi2.at[b], sem_i[b])
            pltpu.async_copy(didx_h.at[pl.ds(base, KC)], di2.at[b], sem_i[b])

        def wait_idx(b):
            pltpu.make_async_copy(gidx_h.at[pl.ds(0, KC)], gi2.at[b],
                                  sem_i[b]).wait()
            pltpu.make_async_copy(didx_h.at[pl.ds(0, KC)], di2.at[b],
                                  sem_i[b]).wait()

        def fire_gather(b):
            wait_idx(b)
            pltpu.async_copy(tables[q].at[gi2.at[b]], rows2.at[b], sem_g[b])

        def drain(b):
            pltpu.make_async_copy(tables[q].at[pl.ds(0, KC)], rows2.at[b],
                                  sem_g[b]).wait()
            pltpu.sync_copy(rows2.at[b], acc_sh.at[di2.at[b]], add=True)

        fire_idx(0, 0)
        fire_idx(1, 1)

        @pl.loop(0, NFULL2 // 2)
        def _(m):
            fire_gather(0)
            fire_gather(1)
            drain(0)
            fire_idx(2 * m + 2, 0)
            drain(1)
            fire_idx(2 * m + 3, 1)

        wait_idx(0)
        wait_idx(1)

        @pl.when(sub < 4)
        def _():
            base = ((TAIL_BASE // KC + sub) * H + kk) * KC
            pltpu.async_copy(gidx_h.at[pl.ds(base, KC)], gi2.at[0], sem_i[0])
            pltpu.async_copy(didx_h.at[pl.ds(base, KC)], di2.at[0], sem_i[0])
            fire_gather(0)
            drain(0)

    for p in range(np2):
        @pl.loop(0, vrow)
        def _(r):
            vbuf[r, pl.ds(0, 16)] = jnp.full((16,), 0.0, jnp.float32)
            vbuf[r, pl.ds(16, 16)] = jnp.full((16,), 0.0, jnp.float32)

        for j in range(4):
            pltpu.sync_copy(vbuf,
                            acc_sh.at[pl.ds(sub * mrow + j * vrow, vrow)])
        plsc.subcore_barrier()
        for c in range(2):
            @pl.when(core == c)
            def _(c=c, p=p):
                sweep(c * np2 + p)
        plsc.subcore_barrier()
        for c in range(2):
            @pl.when(core == c)
            def _(c=c, p=p):
                q = c * np2 + p
                for j in range(4):
                    sl = pl.ds(sub * mrow + j * vrow, vrow)
                    pltpu.sync_copy(acc_sh.at[sl], vbuf)
                    pltpu.sync_copy(vbuf, parts[q].at[sl])
        plsc.subcore_barrier()


def _p2_call(nch, gidx, didx, tables):
    np_ = nch * H
    f = pl.kernel(
        functools.partial(_p2_body, nch),
        out_type=[jax.ShapeDtypeStruct((2 * N, FC), jnp.float32)] * np_,
        mesh=_mesh(),
        scratch_types=[pltpu.VMEM((2, KC), jnp.int32),
                       pltpu.VMEM((2, KC), jnp.int32),
                       pltpu.VMEM((2, KC, FC), jnp.float32),
                       pltpu.VMEM((2 * N // 64, FC), jnp.float32),
                       pltpu.VMEM_SHARED((2 * N, FC), jnp.float32),
                       [pltpu.SemaphoreType.DMA] * 2,
                       [pltpu.SemaphoreType.DMA] * 2],
        compiler_params=pltpu.CompilerParams(use_tc_tiling_on_sc=False),
    )
    return f(gidx, didx, *tables)


def _prep_compute(nch, f, x, w_ref, as_ref, ad_ref, outs):
    np_ = nch * H
    tbl = outs[:np_]
    aso_ref, ado_ref = outs[np_], outs[np_ + 1]
    h = jnp.dot(x, w_ref[...], preferred_element_type=jnp.float32)
    asv = jnp.dot(h, as_ref[...], preferred_element_type=jnp.float32)
    adv = jnp.dot(h, ad_ref[...], preferred_element_type=jnp.float32)
    aso_ref[...] = asv
    ado_ref[...] = adv
    f1 = jnp.exp(asv)
    f2 = jnp.exp(0.2 * asv)
    for p in range(np_):
        k, c = p // nch, p % nch
        cols = h[:, k * f + c * FC: k * f + (c + 1) * FC]
        tbl[p][0] = f1[:, k:k + 1] * cols
        tbl[p][1] = f2[:, k:k + 1] * cols


def _prep_body(nch, f, x_ref, w_ref, as_ref, ad_ref, *outs):
    _prep_compute(nch, f, x_ref[...], w_ref, as_ref, ad_ref, outs)


def _prep_call(nch, f, x, W, asm, adm):
    np_ = nch * H
    cin = x.shape[1]
    c = W.shape[1]
    bm = 512
    grid = (N // bm,)
    return pl.pallas_call(
        functools.partial(_prep_body, nch, f),
        grid=grid,
        in_specs=[pl.BlockSpec((bm, cin), lambda i: (i, 0)),
                  pl.BlockSpec((cin, c), lambda i: (0, 0)),
                  pl.BlockSpec((c, H), lambda i: (0, 0)),
                  pl.BlockSpec((c, H), lambda i: (0, 0))],
        out_specs=[pl.BlockSpec((2, bm, FC), lambda i: (0, i, 0))] * np_
                  + [pl.BlockSpec((bm, H), lambda i: (i, 0))] * 2,
        out_shape=[jax.ShapeDtypeStruct((2, N, FC), jnp.float32)] * np_
                  + [jax.ShapeDtypeStruct((N, H), jnp.float32)] * 2,
    )(x, W, asm, adm)


def _combine_compute(nch, concat, ad, dent, b, parts):
    np_ = nch * H
    f1 = jnp.exp(ad)
    f2 = jnp.exp(0.2 * ad)
    pieces = []
    for p in range(np_):
        k = p // nch
        pr = parts[p]
        num = f1[:, k:k + 1] * pr[0] + f2[:, k:k + 1] * pr[1]
        pieces.append(num / (dent[:, k:k + 1] + 1e-16))
    if concat:
        o = jnp.concatenate(pieces, axis=1)
    else:
        o = sum(pieces) / float(np_)
    return jax.nn.relu(o + b)


def _c1p2_body(ad_ref, den_ref, b_ref, *rest):
    parts = rest[:8]
    w_ref, as_ref, ad2_ref = rest[8:11]
    outs = rest[11:]
    h = _combine_compute(2, True, ad_ref[...], den_ref[0] + den_ref[1],
                         b_ref[...], [p[...] for p in parts])
    _prep_compute(1, 32, h, w_ref, as_ref, ad2_ref, outs)


def _c1p2_call(adv, den, b, parts, W2, asm2, adm2):
    bm = 512
    return pl.pallas_call(
        _c1p2_body,
        grid=(N // bm,),
        in_specs=[pl.BlockSpec((bm, H), lambda i: (i, 0)),
                  pl.BlockSpec((2, bm, H), lambda i: (0, i, 0)),
                  pl.BlockSpec((1, 256), lambda i: (0, 0))]
                 + [pl.BlockSpec((2, bm, FC), lambda i: (0, i, 0))] * 8
                 + [pl.BlockSpec((256, 128), lambda i: (0, 0)),
                    pl.BlockSpec((128, H), lambda i: (0, 0)),
                    pl.BlockSpec((128, H), lambda i: (0, 0))],
        out_specs=[pl.BlockSpec((2, bm, FC), lambda i: (0, i, 0))] * 4
                  + [pl.BlockSpec((bm, H), lambda i: (i, 0))] * 2,
        out_shape=[jax.ShapeDtypeStruct((2, N, FC), jnp.float32)] * 4
                  + [jax.ShapeDtypeStruct((N, H), jnp.float32)] * 2,
    )(adv, den, b.reshape(1, -1), *parts, W2, asm2, adm2)


def _c2_body(ad_ref, den_ref, b_ref, *rest):
    parts = rest[:4]
    out_ref = rest[4]
    out_ref[...] = _combine_compute(1, False, ad_ref[...],
                                    den_ref[0] + den_ref[1], b_ref[...],
                                    [p[...] for p in parts])


def _c2_call(adv, den, b, parts):
    bm = 512
    return pl.pallas_call(
        _c2_body,
        grid=(N // bm,),
        in_specs=[pl.BlockSpec((bm, H), lambda i: (i, 0)),
                  pl.BlockSpec((2, bm, H), lambda i: (0, i, 0)),
                  pl.BlockSpec((1, 32), lambda i: (0, 0))]
                 + [pl.BlockSpec((2, bm, FC), lambda i: (0, i, 0))] * 4,
        out_specs=pl.BlockSpec((bm, 32), lambda i: (i, 0)),
        out_shape=jax.ShapeDtypeStruct((N, 32), jnp.float32),
    )(adv, den, b.reshape(1, -1), *parts)


def _mlp_body(hf_ref, w1_ref, b1_ref, w2_ref, b2_ref, w3_ref, b3_ref, o_ref):
    z = jnp.dot(hf_ref[...], w1_ref[...], preferred_element_type=jnp.float32)
    z = jax.nn.relu(z + b1_ref[...])
    z = jnp.dot(z, w2_ref[...], preferred_element_type=jnp.float32)
    z = jax.nn.relu(z + b2_ref[...])
    z = jnp.dot(z, w3_ref[...], preferred_element_type=jnp.float32)
    o_ref[...] = jax.nn.sigmoid(z + b3_ref[...])


def _mlp_head(hf, lw1, lb1, lw2, lb2, lw3, lb3):
    bs = hf.shape[0]
    return pl.pallas_call(
        _mlp_body,
        in_specs=[pl.BlockSpec(hf.shape, lambda: (0, 0)),
                  pl.BlockSpec(lw1.shape, lambda: (0, 0)),
                  pl.BlockSpec((1, 256), lambda: (0, 0)),
                  pl.BlockSpec(lw2.shape, lambda: (0, 0)),
                  pl.BlockSpec((1, 64), lambda: (0, 0)),
                  pl.BlockSpec(lw3.shape, lambda: (0, 0)),
                  pl.BlockSpec((1, 1), lambda: (0, 0))],
        out_specs=pl.BlockSpec((bs, 1), lambda: (0, 0)),
        out_shape=jax.ShapeDtypeStruct((bs, 1), jnp.float32),
    )(hf, lw1, lb1.reshape(1, -1), lw2, lb2.reshape(1, -1), lw3,
      lb3.reshape(1, -1))


def _diag_mix(a):
    h, f = a.shape
    eye = jnp.eye(h, dtype=a.dtype)
    return (a[:, :, None] * eye[:, None, :]).reshape(h * f, h)


def kernel(x, edge_index, batch, W1, a_src1, a_dst1, b1, W2, a_src2, a_dst2,
           b2, lw1, lb1, lw2, lb2, lw3, lb3):
    loop = jnp.arange(N, dtype=edge_index.dtype)
    src = jnp.concatenate([edge_index[0], loop])
    dst = jnp.concatenate([edge_index[1], loop])

    outs1 = _prep_call(2, 64, x, W1, _diag_mix(a_src1), _diag_mix(a_dst1))
    t1 = [t.reshape(2 * N, FC) for t in outs1[:8]]
    asv1, adv1 = outs1[8], outs1[9]
    gidx1, didx1, den1 = _p1_call(src, dst, asv1.T.reshape(-1),
                                  adv1.T.reshape(-1))
    parts1 = [p.reshape(2, N, FC) for p in _p2_call(2, gidx1, didx1, t1)]
    den1t = den1.reshape(2, H, N).transpose(0, 2, 1)

    outs2 = _c1p2_call(adv1, den1t, b1, parts1, W2,
                       _diag_mix(a_src2), _diag_mix(a_dst2))
    t2 = [t.reshape(2 * N, FC) for t in outs2[:4]]
    asv2, adv2 = outs2[4], outs2[5]
    gidx2, didx2, den2 = _p1_call(src, dst, asv2.T.reshape(-1),
                                  adv2.T.reshape(-1))
    parts2 = [p.reshape(2, N, FC) for p in _p2_call(1, gidx2, didx2, t2)]
    den2t = den2.reshape(2, H, N).transpose(0, 2, 1)

    h2 = _c2_call(adv2, den2t, b2, parts2)
    hf = h2.reshape(N // 33, 33 * 32)
    return _mlp_head(hf, lw1, lb1, lw2, lb2, lw3, lb3)
